# baseline (speedup 1.0000x reference)
import jax
import jax.numpy as jnp
from jax.experimental import pallas as pl
from jax.experimental.pallas import tpu as pltpu

_BH = 4096
_K = 4


def _sub_loss(x, r, mu, mu2):
    x2 = jnp.sum(x * x, axis=1, keepdims=True)
    cross = jax.lax.dot_general(
        x, mu,
        dimension_numbers=(((1,), (1,)), ((), ())),
        preferred_element_type=jnp.float32,
    )
    dist2 = jnp.maximum(x2 + mu2 - 2.0 * cross, 0.0)
    return jnp.sum(r * dist2, axis=0)


def _loss_body(*refs):
    x_refs = refs[:_K]
    r_refs = refs[_K:2 * _K]
    mu_ref = refs[2 * _K]
    o_ref = refs[2 * _K + 1]
    mu = mu_ref[...]
    mu2 = jnp.sum(mu * mu, axis=1, keepdims=True).T
    acc = _sub_loss(x_refs[0][...], r_refs[0][...], mu, mu2)
    for k in range(1, _K):
        acc = acc + _sub_loss(x_refs[k][...], r_refs[k][...], mu, mu2)
    o_ref[0, 0, :] = acc


def kernel(X, r, mus):
    n, d = X.shape
    m = mus.shape[0]
    g = n // (_K * _BH)
    g2 = g // 2

    def _spec(k, w):
        return pl.BlockSpec(
            (_BH, w), lambda i, j, k=k: (_K * (i * g2 + j) + k, 0)
        )

    in_specs = (
        [_spec(k, d) for k in range(_K)]
        + [_spec(k, m) for k in range(_K)]
        + [pl.BlockSpec((m, d), lambda i, j: (0, 0))]
    )
    partials = pl.pallas_call(
        _loss_body,
        grid=(2, g2),
        in_specs=in_specs,
        out_specs=pl.BlockSpec((1, 1, m), lambda i, j: (i * g2 + j, 0, 0)),
        out_shape=jax.ShapeDtypeStruct((g, 1, m), jnp.float32),
        compiler_params=pltpu.CompilerParams(
            dimension_semantics=("parallel", "arbitrary"),
        ),
    )(*([X] * _K + [r] * _K + [mus]))
    return jnp.sum(partials) / n


# speedup vs baseline: 1.0000x; 1.0000x over previous
import jax
import jax.numpy as jnp
from jax.experimental import pallas as pl
from jax.experimental.pallas import tpu as pltpu

_BH = 4096
_K = 2


def _sub_loss(x, r, mu_m2, mu2, ones_dm):
    xx = x * x
    x2bc = jax.lax.dot_general(
        xx, ones_dm,
        dimension_numbers=(((1,), (0,)), ((), ())),
        preferred_element_type=jnp.float32,
    )
    cross_m2 = jax.lax.dot_general(
        x, mu_m2,
        dimension_numbers=(((1,), (1,)), ((), ())),
        preferred_element_type=jnp.float32,
    )
    dist2 = jnp.maximum(x2bc + cross_m2 + mu2, 0.0)
    return jnp.sum(r * dist2, axis=0)


def _loss_body(*refs):
    x_refs = refs[:_K]
    r_refs = refs[_K:2 * _K]
    mu_ref = refs[2 * _K]
    o_ref = refs[2 * _K + 1]
    mu = mu_ref[...]
    mu_m2 = -2.0 * mu
    mu2 = jnp.sum(mu * mu, axis=1, keepdims=True).T
    ones_dm = jnp.ones((mu.shape[1], mu.shape[0]), jnp.float32)
    acc = _sub_loss(x_refs[0][...], r_refs[0][...], mu_m2, mu2, ones_dm)
    for k in range(1, _K):
        acc = acc + _sub_loss(x_refs[k][...], r_refs[k][...], mu_m2, mu2,
                              ones_dm)

    @pl.when(pl.program_id(1) == 0)
    def _init():
        o_ref[0, 0, :] = acc

    @pl.when(pl.program_id(1) != 0)
    def _accum():
        o_ref[0, 0, :] += acc


def kernel(X, r, mus):
    n, d = X.shape
    m = mus.shape[0]
    g = n // (_K * _BH)
    g2 = g // 2

    def _spec(k, w):
        return pl.BlockSpec(
            (_BH, w), lambda i, j, k=k: (_K * (i * g2 + j) + k, 0)
        )

    in_specs = (
        [_spec(k, d) for k in range(_K)]
        + [_spec(k, m) for k in range(_K)]
        + [pl.BlockSpec((m, d), lambda i, j: (0, 0))]
    )
    partials = pl.pallas_call(
        _loss_body,
        grid=(2, g2),
        in_specs=in_specs,
        out_specs=pl.BlockSpec((1, 1, m), lambda i, j: (i, 0, 0)),
        out_shape=jax.ShapeDtypeStruct((2, 1, m), jnp.float32),
        compiler_params=pltpu.CompilerParams(
            dimension_semantics=("parallel", "arbitrary"),
        ),
    )(*([X] * _K + [r] * _K + [mus]))
    return jnp.sum(partials) / n


# speedup vs baseline: 1.0425x; 1.0424x over previous
import jax
import jax.numpy as jnp
from jax.experimental import pallas as pl
from jax.experimental.pallas import tpu as pltpu

_BH = 4096


def _half_loss(x, r, mu, mu2):
    x2 = jnp.sum(x * x, axis=1, keepdims=True)
    cross = jax.lax.dot_general(
        x, mu,
        dimension_numbers=(((1,), (1,)), ((), ())),
        preferred_element_type=jnp.float32,
    )
    dist2 = jnp.maximum(x2 + mu2 - 2.0 * cross, 0.0)
    return jnp.sum(r * dist2, axis=0)


def _loss_body(x0_ref, x1_ref, r0_ref, r1_ref, mu_ref, o_ref):
    mu = mu_ref[...]
    mu2 = jnp.sum(mu * mu, axis=1, keepdims=True).T
    s0 = _half_loss(x0_ref[...], r0_ref[...], mu, mu2)
    s1 = _half_loss(x1_ref[...], r1_ref[...], mu, mu2)
    o_ref[0, 0, :] = s0 + s1


def kernel(X, r, mus):
    n, d = X.shape
    m = mus.shape[0]
    g = n // (2 * _BH)
    g2 = g // 2
    partials = pl.pallas_call(
        _loss_body,
        grid=(2, g2),
        in_specs=[
            pl.BlockSpec((_BH, d), lambda i, j: (2 * (i * g2 + j), 0)),
            pl.BlockSpec((_BH, d), lambda i, j: (2 * (i * g2 + j) + 1, 0)),
            pl.BlockSpec((_BH, m), lambda i, j: (2 * (i * g2 + j), 0)),
            pl.BlockSpec((_BH, m), lambda i, j: (2 * (i * g2 + j) + 1, 0)),
            pl.BlockSpec((m, d), lambda i, j: (0, 0)),
        ],
        out_specs=pl.BlockSpec((1, 1, m), lambda i, j: (i * g2 + j, 0, 0)),
        out_shape=jax.ShapeDtypeStruct((g, 1, m), jnp.float32),
        compiler_params=pltpu.CompilerParams(
            dimension_semantics=("parallel", "arbitrary"),
        ),
    )(X, X, r, r, mus)
    return jnp.sum(partials) / n
